# baseline (speedup 1.0000x reference)
import jax
import jax.numpy as jnp
from jax.experimental import pallas as pl
from jax.experimental.pallas import tpu as pltpu

_INDEX = 2

_MAX_CHUNK_ROWS = 512
_NSLOTS = 4
_DEPTH = 2


def _chunk_schedule(rows):
    head = [64, 64, 128, 256]
    tail = [256, 128, 64, 64]
    middle_rows = rows - sum(head) - sum(tail)
    assert middle_rows >= 0 and middle_rows % _MAX_CHUNK_ROWS == 0
    sizes = head + [_MAX_CHUNK_ROWS] * (middle_rows // _MAX_CHUNK_ROWS) + tail
    chunks = []
    off = 0
    for s in sizes:
        chunks.append((off, s))
        off += s
    return chunks


def _copy_kernel(x_hbm, o_hbm, buf, sem_in, sem_out):
    rows, cols = o_hbm.shape
    chunks = _chunk_schedule(rows)
    nsteps = len(chunks)

    def in_copy(i):
        off, sz = chunks[i]
        return pltpu.make_async_copy(
            x_hbm.at[_INDEX, pl.ds(off, sz), :],
            buf.at[i % _NSLOTS, pl.ds(0, sz), :],
            sem_in.at[i % _NSLOTS],
        )

    def out_copy(i):
        off, sz = chunks[i]
        return pltpu.make_async_copy(
            buf.at[i % _NSLOTS, pl.ds(0, sz), :],
            o_hbm.at[pl.ds(off, sz), :],
            sem_out.at[i % _NSLOTS],
        )

    for i in range(min(_DEPTH, nsteps)):
        in_copy(i).start()
    for i in range(nsteps):
        in_copy(i).wait()
        out_copy(i).start()
        j = i + _DEPTH
        if j < nsteps:
            if j - _NSLOTS >= 0:
                out_copy(j - _NSLOTS).wait()
            in_copy(j).start()
    for i in range(max(0, nsteps - _NSLOTS), nsteps):
        out_copy(i).wait()


def kernel(x):
    _, rows, cols = x.shape
    return pl.pallas_call(
        _copy_kernel,
        out_shape=jax.ShapeDtypeStruct(x.shape[1:], x.dtype),
        in_specs=[pl.BlockSpec(memory_space=pltpu.MemorySpace.HBM)],
        out_specs=pl.BlockSpec(memory_space=pltpu.MemorySpace.HBM),
        scratch_shapes=[
            pltpu.VMEM((_NSLOTS, _MAX_CHUNK_ROWS, cols), x.dtype),
            pltpu.SemaphoreType.DMA((_NSLOTS,)),
            pltpu.SemaphoreType.DMA((_NSLOTS,)),
        ],
        compiler_params=pltpu.CompilerParams(skip_device_barrier=True),
    )(x)
